# baseline (speedup 1.0000x reference)
import functools

import jax
import jax.numpy as jnp
from jax import lax
from jax.experimental import pallas as pl
from jax.experimental.pallas import tpu as pltpu
from jax.experimental.pallas import tpu_sc as plsc

B = 16384
L = 20
D = 128
DH = D // 2
NJ = DH // 16
V = 1000
VPAD = 1024

NC = 2
NS = 16
NW = NC * NS
NRB = NW // 2
CR = 16
NQ = CR // 4
QI = 80
NBUF = 2
OBUF = 2

NSPLIT = 1
BSP = B // NSPLIT


def _make_pool_body(nbatch):
    bpt = nbatch // NRB
    nch = bpt // CR

    def _pool_body(
        idx_hbm, table_hbm, out_hbm,
        table_v, idx0, idx1, outb0, outb1,
        sem_t, semi0, semi1, semo0, semo1,
    ):
        wid = lax.axis_index("s") * NC + lax.axis_index("c")
        half = wid % 2
        rb = wid // 2
        idxb = (idx0, idx1)
        semi = (semi0, semi1)
        outb = (outb0, outb1)
        semo = (semo0, semo1)

        tdma = pltpu.async_copy(table_hbm.at[half], table_v, sem_t)

        def fire_idx(ci, bsel):
            pltpu.async_copy(idx_hbm.at[rb, ci], idxb[bsel], semi[bsel])

        def drain_idx(ci, bsel):
            pltpu.make_async_copy(idx_hbm.at[rb, ci], idxb[bsel], semi[bsel]).wait()

        def out_slice(ci):
            return out_hbm.at[half, pl.ds((rb * bpt + ci * CR) * DH, CR * DH)]

        for p in range(NBUF - 1):
            fire_idx(p, p)
        tdma.wait()

        @pl.loop(0, nch, step=NBUF)
        def _c(ci):
            for bsel in range(NBUF):
                cur = ci + bsel

                @pl.when(cur + NBUF - 1 < nch)
                def _():
                    fire_idx(cur + NBUF - 1, (bsel + NBUF - 1) % NBUF)

                obsel = bsel % OBUF
                drain_idx(cur, bsel)

                @pl.when(cur >= OBUF)
                def _():
                    pltpu.make_async_copy(
                        outb[obsel], out_slice(cur - OBUF), semo[obsel]
                    ).wait()

                @pl.loop(0, NQ)
                def _q(rq):
                    vecs = [
                        idxb[bsel][pl.ds(rq * QI + k * 16, 16)] for k in range(5)
                    ]

                    def gidx(p):
                        return vecs[p // 16][p % 16]

                    for r in range(4):
                        accs = [
                            table_v[gidx(r * L), pl.ds(j * 16, 16)]
                            for j in range(NJ)
                        ]
                        for l in range(1, L):
                            il = gidx(r * L + l)
                            accs = [
                                accs[j] + table_v[il, pl.ds(j * 16, 16)]
                                for j in range(NJ)
                            ]
                        for j in range(NJ):
                            outb[obsel][pl.ds((rq * 4 + r) * DH + j * 16, 16)] = (
                                accs[j] * (1.0 / L)
                            )

                pltpu.async_copy(outb[obsel], out_slice(cur), semo[obsel])

        for p in range(OBUF):
            ci = nch - OBUF + p
            pltpu.make_async_copy(outb[ci % OBUF], out_slice(ci), semo[ci % OBUF]).wait()

    return _pool_body


@functools.cache
def _pool_sc(nbatch):
    return pl.kernel(
        _make_pool_body(nbatch),
        out_type=jax.ShapeDtypeStruct((2, nbatch * DH), jnp.float32),
        mesh=plsc.VectorSubcoreMesh(
            core_axis_name="c", subcore_axis_name="s", num_cores=NC, num_subcores=NS
        ),
        scratch_types=(
            [pltpu.VMEM((V, DH), jnp.float32)]
            + [pltpu.VMEM((NQ * QI,), jnp.int32) for _ in range(NBUF)]
            + [pltpu.VMEM((CR * DH,), jnp.float32) for _ in range(OBUF)]
            + [pltpu.SemaphoreType.DMA for _ in range(NBUF + OBUF + 1)]
        ),
    )


def _argmax_body(x0_ref, x1_ref, w_ref, b_ref, o_ref):
    x = jnp.concatenate([x0_ref[...], x1_ref[...]], axis=1)
    logits = lax.dot_general(
        x, w_ref[...], (((1,), (1,)), ((), ())), preferred_element_type=jnp.float32
    )
    logits = logits + b_ref[...]
    col = lax.broadcasted_iota(jnp.int32, logits.shape, 1)
    m = jnp.max(logits, axis=1, keepdims=True)
    o_ref[...] = jnp.min(jnp.where(logits == m, col, jnp.int32(2**30)), axis=1)


def _argmax_tc(x0, x1, w_pad, b_pad):
    BT = 1024
    nb = x0.shape[0]
    return pl.pallas_call(
        _argmax_body,
        grid=(nb // BT,),
        in_specs=[
            pl.BlockSpec((BT, DH), lambda i: (i, 0)),
            pl.BlockSpec((BT, DH), lambda i: (i, 0)),
            pl.BlockSpec((VPAD, D), lambda i: (0, 0)),
            pl.BlockSpec((1, VPAD), lambda i: (0, 0)),
        ],
        out_specs=pl.BlockSpec((BT,), lambda i: (i,)),
        out_shape=jax.ShapeDtypeStruct((nb,), jnp.int32),
    )(x0, x1, w_pad, b_pad)


@jax.jit
def kernel(input, emb_table, W, b):
    idx = input.astype(jnp.int32).reshape(NSPLIT, NRB, BSP // NRB // CR, NQ * QI)
    table_r = emb_table.reshape(V, 2, DH).transpose(1, 0, 2)
    w_pad = jnp.zeros((VPAD, D), jnp.float32).at[:V].set(W)
    b_pad = jnp.full((1, VPAD), -1e30, jnp.float32).at[0, :V].set(b)

    outs = []
    xh_prev = None
    for s in range(NSPLIT):
        idx_s = idx[s]
        if xh_prev is not None:
            idx_s, _ = lax.optimization_barrier((idx_s, xh_prev))
        xh_prev = _pool_sc(BSP)(idx_s, table_r)
        outs.append(
            _argmax_tc(
                xh_prev[0].reshape(BSP, DH),
                xh_prev[1].reshape(BSP, DH),
                w_pad,
                b_pad,
            )
        )
    return jnp.concatenate(outs)


# speedup vs baseline: 1.0265x; 1.0265x over previous
import functools

import jax
import jax.numpy as jnp
from jax import lax
from jax.experimental import pallas as pl
from jax.experimental.pallas import tpu as pltpu
from jax.experimental.pallas import tpu_sc as plsc

B = 16384
L = 20
D = 128
DH = D // 2
NJ = DH // 16
V = 1000
VPAD = 1024

NC = 2
NS = 16
NW = NC * NS
NRB = NW // 2
CR = 16
NQ = CR // 4
QI = 80
NBUF = 2
OBUF = 2

NSPLIT = 2
BSP = B // NSPLIT


def _make_pool_body(nbatch):
    bpt = nbatch // NRB
    nch = bpt // CR

    def _pool_body(
        idx_hbm, table_hbm, out_hbm,
        table_v, idx0, idx1, outb0, outb1,
        sem_t, semi0, semi1, semo0, semo1,
    ):
        wid = lax.axis_index("s") * NC + lax.axis_index("c")
        half = wid % 2
        rb = wid // 2
        idxb = (idx0, idx1)
        semi = (semi0, semi1)
        outb = (outb0, outb1)
        semo = (semo0, semo1)

        tdma = pltpu.async_copy(table_hbm.at[half], table_v, sem_t)

        def fire_idx(ci, bsel):
            pltpu.async_copy(idx_hbm.at[rb, ci], idxb[bsel], semi[bsel])

        def drain_idx(ci, bsel):
            pltpu.make_async_copy(idx_hbm.at[rb, ci], idxb[bsel], semi[bsel]).wait()

        def out_slice(ci):
            return out_hbm.at[half, pl.ds((rb * bpt + ci * CR) * DH, CR * DH)]

        for p in range(NBUF - 1):
            fire_idx(p, p)
        tdma.wait()

        @pl.loop(0, nch, step=NBUF)
        def _c(ci):
            for bsel in range(NBUF):
                cur = ci + bsel

                @pl.when(cur + NBUF - 1 < nch)
                def _():
                    fire_idx(cur + NBUF - 1, (bsel + NBUF - 1) % NBUF)

                obsel = bsel % OBUF
                drain_idx(cur, bsel)

                @pl.when(cur >= OBUF)
                def _():
                    pltpu.make_async_copy(
                        outb[obsel], out_slice(cur - OBUF), semo[obsel]
                    ).wait()

                @pl.loop(0, NQ)
                def _q(rq):
                    vecs = [
                        idxb[bsel][pl.ds(rq * QI + k * 16, 16)] for k in range(5)
                    ]

                    def gidx(p):
                        return vecs[p // 16][p % 16]

                    for r in range(4):
                        accs = [
                            table_v[gidx(r * L), pl.ds(j * 16, 16)]
                            for j in range(NJ)
                        ]
                        for l in range(1, L):
                            il = gidx(r * L + l)
                            accs = [
                                accs[j] + table_v[il, pl.ds(j * 16, 16)]
                                for j in range(NJ)
                            ]
                        for j in range(NJ):
                            outb[obsel][pl.ds((rq * 4 + r) * DH + j * 16, 16)] = (
                                accs[j] * (1.0 / L)
                            )

                pltpu.async_copy(outb[obsel], out_slice(cur), semo[obsel])

        for p in range(OBUF):
            ci = nch - OBUF + p
            pltpu.make_async_copy(outb[ci % OBUF], out_slice(ci), semo[ci % OBUF]).wait()

    return _pool_body


@functools.cache
def _pool_sc(nbatch):
    return pl.kernel(
        _make_pool_body(nbatch),
        out_type=jax.ShapeDtypeStruct((2, nbatch * DH), jnp.float32),
        mesh=plsc.VectorSubcoreMesh(
            core_axis_name="c", subcore_axis_name="s", num_cores=NC, num_subcores=NS
        ),
        scratch_types=(
            [pltpu.VMEM((V, DH), jnp.float32)]
            + [pltpu.VMEM((NQ * QI,), jnp.int32) for _ in range(NBUF)]
            + [pltpu.VMEM((CR * DH,), jnp.float32) for _ in range(OBUF)]
            + [pltpu.SemaphoreType.DMA for _ in range(NBUF + OBUF + 1)]
        ),
    )


def _argmax_body(x0_ref, x1_ref, w_ref, b_ref, o_ref):
    x = jnp.concatenate([x0_ref[...], x1_ref[...]], axis=1)
    logits = lax.dot_general(
        x, w_ref[...], (((1,), (1,)), ((), ())), preferred_element_type=jnp.float32
    )
    logits = logits + b_ref[...]
    col = lax.broadcasted_iota(jnp.int32, logits.shape, 1)
    m = jnp.max(logits, axis=1, keepdims=True)
    o_ref[...] = jnp.min(jnp.where(logits == m, col, jnp.int32(2**30)), axis=1)


def _argmax_tc(x0, x1, w_pad, b_pad):
    BT = 2048
    nb = x0.shape[0]
    return pl.pallas_call(
        _argmax_body,
        grid=(nb // BT,),
        in_specs=[
            pl.BlockSpec((BT, DH), lambda i: (i, 0)),
            pl.BlockSpec((BT, DH), lambda i: (i, 0)),
            pl.BlockSpec((VPAD, D), lambda i: (0, 0)),
            pl.BlockSpec((1, VPAD), lambda i: (0, 0)),
        ],
        out_specs=pl.BlockSpec((BT,), lambda i: (i,)),
        out_shape=jax.ShapeDtypeStruct((nb,), jnp.int32),
    )(x0, x1, w_pad, b_pad)


@jax.jit
def kernel(input, emb_table, W, b):
    idx = input.astype(jnp.int32).reshape(NSPLIT, NRB, BSP // NRB // CR, NQ * QI)
    table_r = emb_table.reshape(V, 2, DH).transpose(1, 0, 2)
    w_pad = jnp.zeros((VPAD, D), jnp.float32).at[:V].set(W)
    b_pad = jnp.full((1, VPAD), -1e30, jnp.float32).at[0, :V].set(b)

    outs = []
    xh_prev = None
    for s in range(NSPLIT):
        idx_s = idx[s]
        if xh_prev is not None:
            idx_s, _ = lax.optimization_barrier((idx_s, xh_prev))
        xh_prev = _pool_sc(BSP)(idx_s, table_r)
        outs.append(
            _argmax_tc(
                xh_prev[0].reshape(BSP, DH),
                xh_prev[1].reshape(BSP, DH),
                w_pad,
                b_pad,
            )
        )
    return jnp.concatenate(outs)
